# baseline (speedup 1.0000x reference)
import functools

import jax
import jax.numpy as jnp
from jax import lax
from jax.experimental import pallas as pl
from jax.experimental.pallas import tpu as pltpu
from jax.experimental.pallas import tpu_sc as plsc

D_BRAIN = 1024
N_AGENTS = 8
TOK = 8192
BT = 512


_W_CHUNK = 128


def _fused_body(sub_ref, res_ref, w_hbm_ref, rw_ref, kt_ref, rb_ref, mb_ref,
                nw_ref, out_ref, sc_ref, sv_ref, wbf_ref, wstage_ref, wsem):
    t = pl.program_id(0)

    @pl.when(t == 0)
    def _():
        n_ch = (N_AGENTS * D_BRAIN) // _W_CHUNK
        pltpu.make_async_copy(
            w_hbm_ref.at[pl.ds(0, _W_CHUNK)], wstage_ref.at[0], wsem).start()
        for c in range(n_ch):
            pltpu.make_async_copy(
                w_hbm_ref.at[pl.ds(c * _W_CHUNK, _W_CHUNK)],
                wstage_ref.at[c % 2], wsem).wait()
            if c + 1 < n_ch:
                pltpu.make_async_copy(
                    w_hbm_ref.at[pl.ds((c + 1) * _W_CHUNK, _W_CHUNK)],
                    wstage_ref.at[(c + 1) % 2], wsem).start()
            wbf_ref[pl.ds(c * _W_CHUNK, _W_CHUNK), :] = (
                wstage_ref[c % 2].astype(jnp.bfloat16))

    x = sub_ref[...]
    xb = x.astype(jnp.bfloat16)
    q = jnp.dot(xb, rw_ref[...], preferred_element_type=jnp.float32)
    q = q + rb_ref[...]
    scores = jnp.dot(q.astype(jnp.bfloat16), kt_ref[...],
                     preferred_element_type=jnp.float32)
    sc_ref[...] = scores[:, :N_AGENTS]

    @pl.when(t == 0)
    def _():
        lane = lax.broadcasted_iota(jnp.int32, (8, 128), 1)
        row0 = jnp.broadcast_to(scores[0:1, :], (8, 128))
        sv_ref[...] = jnp.where(lane < N_AGENTS, row0, -jnp.inf)

    acc = x + mb_ref[...]
    for a in range(N_AGENTS):
        xa = res_ref[a].astype(jnp.bfloat16)
        wa = wbf_ref[pl.ds(a * D_BRAIN, D_BRAIN), :]
        acc = acc + jnp.dot(xa, wa, preferred_element_type=jnp.float32)
    var = jnp.mean(acc * acc, axis=-1, keepdims=True)
    out_ref[...] = (acc * lax.rsqrt(var + 1e-6)) * nw_ref[...]


@functools.lru_cache(maxsize=1)
def _make_top3_sc_kernel():
    mesh = plsc.VectorSubcoreMesh(core_axis_name="c", subcore_axis_name="s")

    @functools.partial(
        pl.kernel,
        mesh=mesh,
        out_type=jax.ShapeDtypeStruct((16,), jnp.int32),
        scratch_types=[
            pltpu.VMEM((16,), jnp.float32),
            pltpu.VMEM((16,), jnp.int32),
        ],
        compiler_params=pltpu.CompilerParams(needs_layout_passes=False),
    )
    def _top3_sc_kernel(s_hbm, out_hbm, svec, ovec):
        pltpu.sync_copy(s_hbm.at[0, pl.ds(0, 16)], svec)
        idx = lax.iota(jnp.int32, 16)
        _, order = plsc.sort_key_val(svec[...], idx, descending=True)
        ovec[...] = order

        @pl.when(jnp.logical_and(lax.axis_index("c") == 0,
                                 lax.axis_index("s") == 0))
        def _():
            pltpu.sync_copy(ovec, out_hbm)

    return _top3_sc_kernel


def kernel(subtask, results, agent_keys, router_w, router_b, merger_w,
           merger_b, norm_w):
    f32 = jnp.float32
    rw16 = router_w.astype(jnp.bfloat16)
    kt_pad = jnp.zeros((D_BRAIN, 128), jnp.bfloat16)
    kt_pad = kt_pad.at[:, :N_AGENTS].set(agent_keys.T.astype(jnp.bfloat16))
    rb = router_b.reshape(1, D_BRAIN)
    mb = merger_b.reshape(1, D_BRAIN)
    nw = norm_w.reshape(1, D_BRAIN)

    n_t = TOK // BT
    out, scores, svec_arr = pl.pallas_call(
        _fused_body,
        grid=(n_t,),
        in_specs=[
            pl.BlockSpec((BT, D_BRAIN), lambda t: (t, 0)),
            pl.BlockSpec((N_AGENTS, BT, D_BRAIN), lambda t: (0, t, 0)),
            pl.BlockSpec(memory_space=pl.ANY),
            pl.BlockSpec((D_BRAIN, D_BRAIN), lambda t: (0, 0)),
            pl.BlockSpec((D_BRAIN, 128), lambda t: (0, 0)),
            pl.BlockSpec((1, D_BRAIN), lambda t: (0, 0)),
            pl.BlockSpec((1, D_BRAIN), lambda t: (0, 0)),
            pl.BlockSpec((1, D_BRAIN), lambda t: (0, 0)),
        ],
        out_specs=[
            pl.BlockSpec((BT, D_BRAIN), lambda t: (t, 0)),
            pl.BlockSpec((BT, N_AGENTS), lambda t: (t, 0)),
            pl.BlockSpec((8, 128), lambda t: (0, 0)),
        ],
        out_shape=[
            jax.ShapeDtypeStruct((TOK, D_BRAIN), f32),
            jax.ShapeDtypeStruct((TOK, N_AGENTS), f32),
            jax.ShapeDtypeStruct((8, 128), f32),
        ],
        scratch_shapes=[
            pltpu.VMEM((N_AGENTS * D_BRAIN, D_BRAIN), jnp.bfloat16),
            pltpu.VMEM((2, _W_CHUNK, D_BRAIN), f32),
            pltpu.SemaphoreType.DMA,
        ],
        compiler_params=pltpu.CompilerParams(
            dimension_semantics=("arbitrary",),
            vmem_limit_bytes=100 * 1024 * 1024),
    )(subtask, results, merger_w, rw16, kt_pad, rb, mb, nw)

    top3_16 = _make_top3_sc_kernel()(svec_arr)
    top3 = top3_16[:3]
    return (out, scores, top3)


# speedup vs baseline: 1.2211x; 1.2211x over previous
import functools

import jax
import jax.numpy as jnp
from jax import lax
from jax.experimental import pallas as pl
from jax.experimental.pallas import tpu as pltpu
from jax.experimental.pallas import tpu_sc as plsc

D_BRAIN = 1024
N_AGENTS = 8
TOK = 8192
BT = 512


def _fused_body(sub_ref, res_ref, w_ref, rw_ref, kt_ref, rb_ref, mb_ref,
                nw_ref, out_ref, sc_ref, sv_ref):
    t = pl.program_id(0)
    x = sub_ref[...]
    xb = x.astype(jnp.bfloat16)
    q = jnp.dot(xb, rw_ref[...], preferred_element_type=jnp.float32)
    q = q + rb_ref[...]
    scores = jnp.dot(q.astype(jnp.bfloat16), kt_ref[...],
                     preferred_element_type=jnp.float32)
    sc_ref[...] = scores

    @pl.when(t == 0)
    def _():
        lane = lax.broadcasted_iota(jnp.int32, (8, 128), 1)
        row0 = jnp.broadcast_to(scores[0:1, :], (8, 128))
        sv_ref[...] = jnp.where(lane < N_AGENTS, row0, -jnp.inf)

    acc = x + mb_ref[...]
    for a in range(N_AGENTS):
        xa = res_ref[a].astype(jnp.bfloat16)
        acc = acc + jnp.dot(xa, w_ref[a], preferred_element_type=jnp.float32)
    var = jnp.mean(acc * acc, axis=-1, keepdims=True)
    out_ref[...] = (acc * lax.rsqrt(var + 1e-6)) * nw_ref[...]


@functools.lru_cache(maxsize=1)
def _make_top3_sc_kernel():
    mesh = plsc.VectorSubcoreMesh(core_axis_name="c", subcore_axis_name="s")

    @functools.partial(
        pl.kernel,
        mesh=mesh,
        out_type=jax.ShapeDtypeStruct((16,), jnp.int32),
        scratch_types=[
            pltpu.VMEM((16,), jnp.float32),
            pltpu.VMEM((16,), jnp.int32),
        ],
        compiler_params=pltpu.CompilerParams(needs_layout_passes=False),
    )
    def _top3_sc_kernel(s_hbm, out_hbm, svec, ovec):
        pltpu.sync_copy(s_hbm.at[0, pl.ds(0, 16)], svec)
        idx = lax.iota(jnp.int32, 16)
        _, order = plsc.sort_key_val(svec[...], idx, descending=True)
        ovec[...] = order

        @pl.when(jnp.logical_and(lax.axis_index("c") == 0,
                                 lax.axis_index("s") == 0))
        def _():
            pltpu.sync_copy(ovec, out_hbm)

    return _top3_sc_kernel


def kernel(subtask, results, agent_keys, router_w, router_b, merger_w,
           merger_b, norm_w):
    f32 = jnp.float32
    rw16 = router_w.astype(jnp.bfloat16)
    kt_pad = jnp.zeros((D_BRAIN, 128), jnp.bfloat16)
    kt_pad = kt_pad.at[:, :N_AGENTS].set(agent_keys.T.astype(jnp.bfloat16))
    rb = router_b.reshape(1, D_BRAIN)
    w3 = merger_w.astype(jnp.bfloat16).reshape(N_AGENTS, D_BRAIN, D_BRAIN)
    mb = merger_b.reshape(1, D_BRAIN)
    nw = norm_w.reshape(1, D_BRAIN)

    n_t = TOK // BT
    out, scores_pad, svec_arr = pl.pallas_call(
        _fused_body,
        grid=(n_t,),
        in_specs=[
            pl.BlockSpec((BT, D_BRAIN), lambda t: (t, 0)),
            pl.BlockSpec((N_AGENTS, BT, D_BRAIN), lambda t: (0, t, 0)),
            pl.BlockSpec((N_AGENTS, D_BRAIN, D_BRAIN), lambda t: (0, 0, 0)),
            pl.BlockSpec((D_BRAIN, D_BRAIN), lambda t: (0, 0)),
            pl.BlockSpec((D_BRAIN, 128), lambda t: (0, 0)),
            pl.BlockSpec((1, D_BRAIN), lambda t: (0, 0)),
            pl.BlockSpec((1, D_BRAIN), lambda t: (0, 0)),
            pl.BlockSpec((1, D_BRAIN), lambda t: (0, 0)),
        ],
        out_specs=[
            pl.BlockSpec((BT, D_BRAIN), lambda t: (t, 0)),
            pl.BlockSpec((BT, 128), lambda t: (t, 0)),
            pl.BlockSpec((8, 128), lambda t: (0, 0)),
        ],
        out_shape=[
            jax.ShapeDtypeStruct((TOK, D_BRAIN), f32),
            jax.ShapeDtypeStruct((TOK, 128), f32),
            jax.ShapeDtypeStruct((8, 128), f32),
        ],
        compiler_params=pltpu.CompilerParams(
            dimension_semantics=("parallel",),
            vmem_limit_bytes=100 * 1024 * 1024),
    )(subtask, results, w3, rw16, kt_pad, rb, mb, nw)
    scores = scores_pad[:, :N_AGENTS]

    top3_16 = _make_top3_sc_kernel()(svec_arr)
    top3 = top3_16[:3]
    return (out, scores, top3)


# speedup vs baseline: 1.2314x; 1.0084x over previous
import functools

import jax
import jax.numpy as jnp
from jax import lax
from jax.experimental import pallas as pl
from jax.experimental.pallas import tpu as pltpu
from jax.experimental.pallas import tpu_sc as plsc

D_BRAIN = 1024
N_AGENTS = 8
TOK = 8192
BT = 512


def _fused_body(sub_ref, res_ref, w_ref, rw_ref, kt_ref, rb_ref, mb_ref,
                nw_ref, out_ref, sc_ref):
    x = sub_ref[...]
    xb = x.astype(jnp.bfloat16)
    q = jnp.dot(xb, rw_ref[...], preferred_element_type=jnp.float32)
    q = q + rb_ref[...]
    sc_ref[...] = jnp.dot(q.astype(jnp.bfloat16), kt_ref[...],
                          preferred_element_type=jnp.float32)

    acc = x + mb_ref[...]
    for a in range(N_AGENTS):
        xa = res_ref[a].astype(jnp.bfloat16)
        acc = acc + jnp.dot(xa, w_ref[a], preferred_element_type=jnp.float32)
    var = jnp.mean(acc * acc, axis=-1, keepdims=True)
    out_ref[...] = (acc * lax.rsqrt(var + 1e-6)) * nw_ref[...]


@functools.lru_cache(maxsize=1)
def _make_top3_sc_kernel():
    mesh = plsc.VectorSubcoreMesh(core_axis_name="c", subcore_axis_name="s")

    @functools.partial(
        pl.kernel,
        mesh=mesh,
        out_type=jax.ShapeDtypeStruct((16,), jnp.int32),
        scratch_types=[
            pltpu.VMEM((16,), jnp.float32),
            pltpu.VMEM((16,), jnp.int32),
        ],
        compiler_params=pltpu.CompilerParams(needs_layout_passes=False),
    )
    def _top3_sc_kernel(s_hbm, out_hbm, svec, ovec):
        pltpu.sync_copy(s_hbm, svec)
        idx = lax.iota(jnp.int32, 16)
        _, order = plsc.sort_key_val(svec[...], idx, descending=True)
        ovec[...] = order

        @pl.when(jnp.logical_and(lax.axis_index("c") == 0,
                                 lax.axis_index("s") == 0))
        def _():
            pltpu.sync_copy(ovec, out_hbm)

    return _top3_sc_kernel


def kernel(subtask, results, agent_keys, router_w, router_b, merger_w,
           merger_b, norm_w):
    f32 = jnp.float32
    rw16 = router_w.astype(jnp.bfloat16)
    kt_pad = jnp.zeros((D_BRAIN, 128), jnp.bfloat16)
    kt_pad = kt_pad.at[:, :N_AGENTS].set(agent_keys.T.astype(jnp.bfloat16))
    rb = router_b.reshape(1, D_BRAIN)
    w3 = merger_w.astype(jnp.bfloat16).reshape(N_AGENTS, D_BRAIN, D_BRAIN)
    mb = merger_b.reshape(1, D_BRAIN)
    nw = norm_w.reshape(1, D_BRAIN)

    n_t = TOK // BT
    out, scores_pad = pl.pallas_call(
        _fused_body,
        grid=(n_t,),
        in_specs=[
            pl.BlockSpec((BT, D_BRAIN), lambda t: (t, 0)),
            pl.BlockSpec((N_AGENTS, BT, D_BRAIN), lambda t: (0, t, 0)),
            pl.BlockSpec((N_AGENTS, D_BRAIN, D_BRAIN), lambda t: (0, 0, 0)),
            pl.BlockSpec((D_BRAIN, D_BRAIN), lambda t: (0, 0)),
            pl.BlockSpec((D_BRAIN, 128), lambda t: (0, 0)),
            pl.BlockSpec((1, D_BRAIN), lambda t: (0, 0)),
            pl.BlockSpec((1, D_BRAIN), lambda t: (0, 0)),
            pl.BlockSpec((1, D_BRAIN), lambda t: (0, 0)),
        ],
        out_specs=[
            pl.BlockSpec((BT, D_BRAIN), lambda t: (t, 0)),
            pl.BlockSpec((BT, 128), lambda t: (t, 0)),
        ],
        out_shape=[
            jax.ShapeDtypeStruct((TOK, D_BRAIN), f32),
            jax.ShapeDtypeStruct((TOK, 128), f32),
        ],
        compiler_params=pltpu.CompilerParams(
            dimension_semantics=("parallel",),
            vmem_limit_bytes=100 * 1024 * 1024),
    )(subtask, results, w3, rw16, kt_pad, rb, mb, nw)
    scores = scores_pad[:, :N_AGENTS]

    svec_in = jnp.concatenate(
        [scores_pad[0, :N_AGENTS], jnp.full((8,), -jnp.inf, f32)])
    top3_16 = _make_top3_sc_kernel()(svec_in)
    top3 = top3_16[:3]
    return (out, scores, top3)
